# baseline (speedup 1.0000x reference)
import functools

import jax
import jax.numpy as jnp
from jax import lax
from jax.experimental import pallas as pl
from jax.experimental.pallas import tpu as pltpu
from jax.experimental.pallas import tpu_sc as plsc

D_MODEL = 64
BLK = 128
NUM_WORKERS = 32


def _make_kernel(batch, seq):
    assert batch == BLK * NUM_WORKERS
    mesh = plsc.VectorSubcoreMesh(core_axis_name="c", subcore_axis_name="s")

    @functools.partial(
        pl.kernel,
        mesh=mesh,
        out_type=jax.ShapeDtypeStruct(
            (seq, D_MODEL // 8, NUM_WORKERS, 8 * BLK), jnp.float32
        ),
        scratch_types=[
            pltpu.VMEM((BLK, seq + 1), jnp.int32),
            pltpu.VMEM((seq, BLK), jnp.int32),
            pltpu.VMEM((BLK, D_MODEL + 1), jnp.float32),
            [pltpu.VMEM((BLK, D_MODEL), jnp.float32) for _ in range(2)],
            [pltpu.VMEM((D_MODEL // 8, 8 * BLK), jnp.float32) for _ in range(2)],
            [pltpu.SemaphoreType.DMA for _ in range(2)],
            [pltpu.SemaphoreType.DMA for _ in range(2)],
        ],
        compiler_params=pltpu.CompilerParams(
            use_tc_tiling_on_sc=False, needs_layout_passes=False
        ),
    )
    def gather_kernel(
        tok_hbm, table_hbm, y_hbm, idx_raw, idx_t, rows_p, rows, yb, sg, so
    ):
        wid = lax.axis_index("s") * 2 + lax.axis_index("c")
        pltpu.sync_copy(
            tok_hbm.at[pl.ds(wid * BLK, BLK), :], idx_raw.at[:, pl.ds(0, seq)]
        )

        iota = lax.iota(jnp.int32, 16)
        lanes = [iota + (16 * g) for g in range(BLK // 16)]

        @plsc.parallel_loop(0, seq, unroll=4)
        def _idx_transpose(s):
            col = jnp.zeros((16,), jnp.int32) + s
            for g in range(BLK // 16):
                v = plsc.load_gather(idx_raw, [lanes[g], col])
                idx_t[s, pl.ds(16 * g, 16)] = v


        def issue_gather(s, b):
            pltpu.async_copy(table_hbm.at[idx_t.at[s]], rows[b], sg[b])

        def wait_gather(b):
            pltpu.make_async_copy(
                table_hbm.at[pl.ds(0, BLK)], rows[b], sg[b]
            ).wait()

        def transpose(b):
            @plsc.parallel_loop(0, BLK, unroll=8)
            def _repitch(t):
                for d0 in range(0, D_MODEL, 16):
                    rows_p[t, pl.ds(d0, 16)] = rows[b][t, pl.ds(d0, 16)]

            @plsc.parallel_loop(0, D_MODEL, unroll=4)
            def _transpose(d):
                col = jnp.zeros((16,), jnp.int32) + d
                row = lax.shift_right_logical(d, 3)
                off = lax.shift_left(jnp.bitwise_and(d, 7), 7)
                for g in range(BLK // 16):
                    v = plsc.load_gather(rows_p, [lanes[g], col])
                    yb[b][row, pl.ds(off + 16 * g, 16)] = v

        def issue_store(s, b):
            pltpu.async_copy(yb[b], y_hbm.at[s, :, wid, :], so[b])

        def wait_store(b):
            pltpu.make_async_copy(
                yb[b], y_hbm.at[0, :, wid, :], so[b]
            ).wait()

        issue_gather(0, 0)
        issue_gather(1, 1)
        wait_gather(0)
        transpose(0)
        issue_gather(2, 0)
        issue_store(0, 0)
        wait_gather(1)
        transpose(1)
        issue_gather(3, 1)
        issue_store(1, 1)

        def body(i, carry):
            s0 = 2 * i
            wait_gather(0)
            wait_store(0)
            transpose(0)
            pl.when(i < seq // 2 - 1)(lambda: issue_gather(s0 + 2, 0))
            issue_store(s0, 0)
            wait_gather(1)
            wait_store(1)
            transpose(1)
            pl.when(i < seq // 2 - 1)(lambda: issue_gather(s0 + 3, 1))
            issue_store(s0 + 1, 1)
            return carry

        lax.fori_loop(1, seq // 2, body, 0)
        wait_store(0)
        wait_store(1)

    return gather_kernel


def _make_detile(vocab):
    n_full = vocab // 128
    tail = vocab - n_full * 128
    mesh = plsc.VectorSubcoreMesh(core_axis_name="c", subcore_axis_name="s")

    @functools.partial(
        pl.kernel,
        mesh=mesh,
        out_type=jax.ShapeDtypeStruct((vocab * D_MODEL // 128, 128), jnp.float32),
        scratch_types=[
            [pltpu.VMEM((D_MODEL, 130), jnp.float32) for _ in range(2)],
            [pltpu.VMEM((D_MODEL, 128), jnp.float32) for _ in range(2)],
            [pltpu.SemaphoreType.DMA for _ in range(2)],
            [pltpu.SemaphoreType.DMA for _ in range(2)],
        ],
        compiler_params=pltpu.CompilerParams(
            use_tc_tiling_on_sc=True, needs_layout_passes=False
        ),
    )
    def detile_kernel(tt_hbm, tail_hbm, out_hbm, inb, outb, si, so):
        wid = lax.axis_index("s") * 2 + lax.axis_index("c")
        nc = (n_full - wid + NUM_WORKERS - 1) // NUM_WORKERS
        iota = lax.iota(jnp.int32, 16)

        def col(k):
            return wid + NUM_WORKERS * k

        def issue_in(k, b):
            off = pl.multiple_of(col(k) * 128, 128)
            pltpu.async_copy(
                tt_hbm.at[:, pl.ds(off, 128)], inb[b].at[:, pl.ds(0, 128)], si[b]
            )

        def wait_in(b):
            pltpu.make_async_copy(
                tt_hbm.at[:, pl.ds(0, 128)], inb[b].at[:, pl.ds(0, 128)], si[b]
            ).wait()

        def transpose(b, nv):
            @plsc.parallel_loop(0, nv, unroll=8)
            def _t(v):
                cv = jnp.zeros((16,), jnp.int32) + v
                row = lax.shift_right_logical(v, 1)
                base = lax.shift_left(jnp.bitwise_and(v, 1), 6)
                for d0 in range(0, D_MODEL, 16):
                    val = plsc.load_gather(inb[b], [d0 + iota, cv])
                    outb[b][row, pl.ds(base + d0, 16)] = val

        def issue_out(k, b):
            off = pl.multiple_of(col(k) * D_MODEL, 8)
            pltpu.async_copy(outb[b], out_hbm.at[pl.ds(off, D_MODEL), :], so[b])

        def wait_out(b):
            pltpu.make_async_copy(
                outb[b], out_hbm.at[pl.ds(0, D_MODEL), :], so[b]
            ).wait()

        issue_in(0, 0)

        def step(k, b):
            pl.when(k + 1 < nc)(lambda: issue_in(k + 1, 1 - b))
            wait_in(b)
            pl.when(k >= 2)(lambda: wait_out(b))
            transpose(b, 128)
            issue_out(k, b)

        def body2(j, carry):
            k0 = 2 * j
            pl.when(k0 < nc)(lambda: step(k0, 0))
            pl.when(k0 + 1 < nc)(lambda: step(k0 + 1, 1))
            return carry

        lax.fori_loop(0, (n_full + NUM_WORKERS - 1) // NUM_WORKERS // 2 + 1, body2, 0)
        pl.when(nc >= 2)(lambda: wait_out(0))
        pl.when(nc >= 1)(lambda: wait_out(1))

        if tail:
            tail_rows = tail * D_MODEL // 128

            @pl.when(wid == 0)
            def _tail():
                pltpu.sync_copy(tail_hbm, outb[0].at[pl.ds(0, tail_rows), :])
                pltpu.sync_copy(
                    outb[0].at[pl.ds(0, tail_rows), :],
                    out_hbm.at[pl.ds(n_full * D_MODEL, tail_rows), :],
                )

    return detile_kernel


def kernel(tokens, token_emb):
    batch, seq = tokens.shape
    vocab = token_emb.shape[0]
    tok = tokens.astype(jnp.int32)
    n_full = vocab // 128
    tail = vocab - n_full * 128
    tail_packed = token_emb[n_full * 128 :].reshape(tail * D_MODEL // 128, 128)
    tbl_packed = _make_detile(vocab)(token_emb.T, tail_packed)
    tbl_lin = tbl_packed.reshape(vocab, D_MODEL)
    y4 = _make_kernel(batch, seq)(tok, tbl_lin)
    y5 = y4.reshape(seq, D_MODEL // 8, NUM_WORKERS, 8, BLK)
    return y5.transpose(2, 4, 0, 1, 3).reshape(batch, seq, D_MODEL)


# speedup vs baseline: 1.0019x; 1.0019x over previous
import functools

import jax
import jax.numpy as jnp
from jax import lax
from jax.experimental import pallas as pl
from jax.experimental.pallas import tpu as pltpu
from jax.experimental.pallas import tpu_sc as plsc

D_MODEL = 64
BLK = 128
NUM_WORKERS = 32


def _make_kernel(batch, seq):
    assert batch == BLK * NUM_WORKERS
    mesh = plsc.VectorSubcoreMesh(core_axis_name="c", subcore_axis_name="s")

    @functools.partial(
        pl.kernel,
        mesh=mesh,
        out_type=jax.ShapeDtypeStruct(
            (seq, D_MODEL // 8, NUM_WORKERS, 8 * BLK), jnp.float32
        ),
        scratch_types=[
            pltpu.VMEM((BLK, seq + 1), jnp.int32),
            pltpu.VMEM((seq, BLK), jnp.int32),
            pltpu.VMEM((BLK, D_MODEL + 1), jnp.float32),
            [pltpu.VMEM((BLK, D_MODEL), jnp.float32) for _ in range(2)],
            [pltpu.VMEM((D_MODEL // 8, 8 * BLK), jnp.float32) for _ in range(2)],
            [pltpu.SemaphoreType.DMA for _ in range(2)],
            [pltpu.SemaphoreType.DMA for _ in range(2)],
        ],
        compiler_params=pltpu.CompilerParams(
            use_tc_tiling_on_sc=False, needs_layout_passes=False
        ),
    )
    def gather_kernel(
        tok_hbm, table_hbm, y_hbm, idx_raw, idx_t, rows_p, rows, yb, sg, so
    ):
        wid = lax.axis_index("s") * 2 + lax.axis_index("c")
        pltpu.sync_copy(
            tok_hbm.at[pl.ds(wid * BLK, BLK), :], idx_raw.at[:, pl.ds(0, seq)]
        )

        iota = lax.iota(jnp.int32, 16)
        lanes = [iota + (16 * g) for g in range(BLK // 16)]

        @plsc.parallel_loop(0, seq, unroll=4)
        def _idx_transpose(s):
            col = jnp.zeros((16,), jnp.int32) + s
            for g in range(BLK // 16):
                v = plsc.load_gather(idx_raw, [lanes[g], col])
                idx_t[s, pl.ds(16 * g, 16)] = v


        def issue_gather(s, b):
            pltpu.async_copy(table_hbm.at[idx_t.at[s]], rows[b], sg[b])

        def wait_gather(b):
            pltpu.make_async_copy(
                table_hbm.at[pl.ds(0, BLK)], rows[b], sg[b]
            ).wait()

        def transpose(b):
            @plsc.parallel_loop(0, BLK, unroll=8)
            def _repitch(t):
                for d0 in range(0, D_MODEL, 16):
                    rows_p[t, pl.ds(d0, 16)] = rows[b][t, pl.ds(d0, 16)]

            @plsc.parallel_loop(0, D_MODEL, unroll=4)
            def _transpose(d):
                col = jnp.zeros((16,), jnp.int32) + d
                row = lax.shift_right_logical(d, 3)
                off = lax.shift_left(jnp.bitwise_and(d, 7), 7)
                for g in range(BLK // 16):
                    v = plsc.load_gather(rows_p, [lanes[g], col])
                    yb[b][row, pl.ds(off + 16 * g, 16)] = v

        def issue_store(s, b):
            pltpu.async_copy(yb[b], y_hbm.at[s, :, wid, :], so[b])

        def wait_store(b):
            pltpu.make_async_copy(
                yb[b], y_hbm.at[0, :, wid, :], so[b]
            ).wait()

        issue_gather(0, 0)
        issue_gather(1, 1)
        wait_gather(0)
        transpose(0)
        issue_gather(2, 0)
        issue_store(0, 0)
        wait_gather(1)
        transpose(1)
        issue_gather(3, 1)
        issue_store(1, 1)

        def body(i, carry):
            s0 = 2 * i
            wait_gather(0)
            wait_store(0)
            transpose(0)
            pl.when(i < seq // 2 - 1)(lambda: issue_gather(s0 + 2, 0))
            issue_store(s0, 0)
            wait_gather(1)
            wait_store(1)
            transpose(1)
            pl.when(i < seq // 2 - 1)(lambda: issue_gather(s0 + 3, 1))
            issue_store(s0 + 1, 1)
            return carry

        lax.fori_loop(1, seq // 2, body, 0)
        wait_store(0)
        wait_store(1)

    return gather_kernel


def _make_detile(vocab):
    n_full = vocab // 128
    tail = vocab - n_full * 128
    mesh = plsc.VectorSubcoreMesh(core_axis_name="c", subcore_axis_name="s")

    @functools.partial(
        pl.kernel,
        mesh=mesh,
        out_type=jax.ShapeDtypeStruct((vocab * D_MODEL // 128, 128), jnp.float32),
        scratch_types=[
            [pltpu.VMEM((D_MODEL, 131), jnp.float32) for _ in range(2)],
            [pltpu.VMEM((D_MODEL, 128), jnp.float32) for _ in range(2)],
            [pltpu.SemaphoreType.DMA for _ in range(2)],
            [pltpu.SemaphoreType.DMA for _ in range(2)],
        ],
        compiler_params=pltpu.CompilerParams(
            use_tc_tiling_on_sc=True, needs_layout_passes=False
        ),
    )
    def detile_kernel(tt_hbm, tail_hbm, out_hbm, inb, outb, si, so):
        wid = lax.axis_index("s") * 2 + lax.axis_index("c")
        nc = (n_full - wid + NUM_WORKERS - 1) // NUM_WORKERS
        iota = lax.iota(jnp.int32, 16)

        def col(k):
            return wid + NUM_WORKERS * k

        def issue_in(k, b):
            off = pl.multiple_of(col(k) * 128, 128)
            pltpu.async_copy(
                tt_hbm.at[:, pl.ds(off, 128)], inb[b].at[:, pl.ds(0, 128)], si[b]
            )

        def wait_in(b):
            pltpu.make_async_copy(
                tt_hbm.at[:, pl.ds(0, 128)], inb[b].at[:, pl.ds(0, 128)], si[b]
            ).wait()

        def transpose(b, nv):
            @plsc.parallel_loop(0, nv, unroll=8)
            def _t(v):
                cv = jnp.zeros((16,), jnp.int32) + v
                row = lax.shift_right_logical(v, 1)
                base = lax.shift_left(jnp.bitwise_and(v, 1), 6)
                for d0 in range(0, D_MODEL, 16):
                    val = plsc.load_gather(inb[b], [d0 + iota, cv])
                    outb[b][row, pl.ds(base + d0, 16)] = val

        def issue_out(k, b):
            off = pl.multiple_of(col(k) * D_MODEL, 8)
            pltpu.async_copy(outb[b], out_hbm.at[pl.ds(off, D_MODEL), :], so[b])

        def wait_out(b):
            pltpu.make_async_copy(
                outb[b], out_hbm.at[pl.ds(0, D_MODEL), :], so[b]
            ).wait()

        issue_in(0, 0)

        def step(k, b):
            pl.when(k + 1 < nc)(lambda: issue_in(k + 1, 1 - b))
            wait_in(b)
            pl.when(k >= 2)(lambda: wait_out(b))
            transpose(b, 128)
            issue_out(k, b)

        def body2(j, carry):
            k0 = 2 * j
            pl.when(k0 < nc)(lambda: step(k0, 0))
            pl.when(k0 + 1 < nc)(lambda: step(k0 + 1, 1))
            return carry

        lax.fori_loop(0, (n_full + NUM_WORKERS - 1) // NUM_WORKERS // 2 + 1, body2, 0)
        pl.when(nc >= 2)(lambda: wait_out(0))
        pl.when(nc >= 1)(lambda: wait_out(1))

        if tail:
            tail_rows = tail * D_MODEL // 128

            @pl.when(wid == 0)
            def _tail():
                pltpu.sync_copy(tail_hbm, outb[0].at[pl.ds(0, tail_rows), :])
                pltpu.sync_copy(
                    outb[0].at[pl.ds(0, tail_rows), :],
                    out_hbm.at[pl.ds(n_full * D_MODEL, tail_rows), :],
                )

    return detile_kernel


def kernel(tokens, token_emb):
    batch, seq = tokens.shape
    vocab = token_emb.shape[0]
    tok = tokens.astype(jnp.int32)
    n_full = vocab // 128
    tail = vocab - n_full * 128
    tail_packed = token_emb[n_full * 128 :].reshape(tail * D_MODEL // 128, 128)
    tbl_packed = _make_detile(vocab)(token_emb.T, tail_packed)
    tbl_lin = tbl_packed.reshape(vocab, D_MODEL)
    y4 = _make_kernel(batch, seq)(tok, tbl_lin)
    y5 = y4.reshape(seq, D_MODEL // 8, NUM_WORKERS, 8, BLK)
    return y5.transpose(2, 4, 0, 1, 3).reshape(batch, seq, D_MODEL)
